# Initial kernel scaffold; baseline (speedup 1.0000x reference)
import functools

import jax
import jax.numpy as jnp
from jax import lax
from jax.experimental import pallas as pl
from jax.experimental.pallas import tpu as pltpu
from jax.experimental.pallas import tpu_sc as plsc

_NC = 2
_NS = 16
_L = 16


def _sc_pool(x_flat, table, batch, seq, dim):
    nw = _NC * _NS
    rows_per_w = batch // nw
    cb = 8
    steps = rows_per_w // cb
    inv_seq = 1.0 / seq
    mesh = plsc.VectorSubcoreMesh(core_axis_name="c", subcore_axis_name="s")

    @functools.partial(
        pl.kernel,
        mesh=mesh,
        out_type=jax.ShapeDtypeStruct((batch, dim), jnp.float32),
        scratch_types=[
            pltpu.VMEM((cb * seq,), jnp.int32),
            pltpu.VMEM((cb * seq, dim), jnp.float32),
            pltpu.VMEM((cb, dim), jnp.float32),
            pltpu.SemaphoreType.DMA,
        ],
    )
    def pool_kernel(x_hbm, table_hbm, out_hbm, idx_v, rows_v, out_v, sem):
        wid = lax.axis_index("s") * _NC + lax.axis_index("c")
        base = wid * rows_per_w

        @pl.loop(0, steps)
        def _(g):
            row0 = base + g * cb
            pltpu.sync_copy(x_hbm.at[pl.ds(row0 * seq, cb * seq)], idx_v)
            pltpu.async_copy(table_hbm.at[idx_v], rows_v, sem).wait()
            for c in range(cb):
                def body(s, accs, c=c):
                    r = c * seq + s
                    return tuple(
                        accs[d] + rows_v[r, pl.ds(d * _L, _L)]
                        for d in range(dim // _L)
                    )
                accs = lax.fori_loop(
                    0, seq, body,
                    tuple(jnp.zeros((_L,), jnp.float32)
                          for _ in range(dim // _L)),
                )
                for d in range(dim // _L):
                    out_v[c, pl.ds(d * _L, _L)] = accs[d] * inv_seq
            pltpu.sync_copy(out_v, out_hbm.at[pl.ds(row0, cb)])

    return pool_kernel(x_flat, table)


def _tc_head(pooled, w, b2d):
    batch, dim = pooled.shape
    seq = w.shape[0]
    bb = 2048

    def head_kernel(p_ref, w_ref, b_ref, o_ref):
        logits = lax.dot_general(
            p_ref[...], w_ref[...],
            (((1,), (1,)), ((), ())),
            preferred_element_type=jnp.float32,
        ) + b_ref[...]
        o_ref[...] = 1.0 / (1.0 + jnp.exp(-logits))

    return pl.pallas_call(
        head_kernel,
        grid=(batch // bb,),
        in_specs=[
            pl.BlockSpec((bb, dim), lambda i: (i, 0)),
            pl.BlockSpec((seq, dim), lambda i: (0, 0)),
            pl.BlockSpec((1, seq), lambda i: (0, 0)),
        ],
        out_specs=pl.BlockSpec((bb, seq), lambda i: (i, 0)),
        out_shape=jax.ShapeDtypeStruct((batch, seq), jnp.float32),
    )(pooled, w, b2d)


@jax.jit
def kernel(x, table, W, b):
    batch, seq = x.shape
    dim = table.shape[1]
    pooled = _sc_pool(x.reshape(batch * seq), table, batch, seq, dim)
    return _tc_head(pooled, W, b.reshape(1, seq))


# speedup vs baseline: 2.3154x; 2.3154x over previous
import functools

import jax
import jax.numpy as jnp
from jax import lax
from jax.experimental import pallas as pl
from jax.experimental.pallas import tpu as pltpu
from jax.experimental.pallas import tpu_sc as plsc

_NC = 2
_NS = 16
_L = 16


def _sc_pool(x_flat, table, batch, seq, dim):
    nw = _NC * _NS
    rows_per_w = batch // nw
    cb = 8
    steps = rows_per_w // cb
    inv_seq = 1.0 / seq
    mesh = plsc.VectorSubcoreMesh(core_axis_name="c", subcore_axis_name="s")

    @functools.partial(
        pl.kernel,
        mesh=mesh,
        compiler_params=pltpu.CompilerParams(use_tc_tiling_on_sc=False),
        out_type=jax.ShapeDtypeStruct((batch, dim), jnp.float32),
        scratch_types=[
            pltpu.VMEM((cb * seq,), jnp.int32),
            pltpu.VMEM((cb * seq, dim), jnp.float32),
            pltpu.VMEM((cb, dim), jnp.float32),
            pltpu.SemaphoreType.DMA,
        ],
    )
    def pool_kernel(x_hbm, table_hbm, out_hbm, idx_v, rows_v, out_v, sem):
        wid = lax.axis_index("s") * _NC + lax.axis_index("c")
        base = wid * rows_per_w

        @pl.loop(0, steps)
        def _(g):
            row0 = base + g * cb
            pltpu.sync_copy(x_hbm.at[pl.ds(row0 * seq, cb * seq)], idx_v)
            pltpu.async_copy(table_hbm.at[idx_v], rows_v, sem).wait()
            for c in range(cb):
                def body(s, accs, c=c):
                    r = c * seq + s
                    return tuple(
                        accs[d] + rows_v[r, pl.ds(d * _L, _L)]
                        for d in range(dim // _L)
                    )
                accs = lax.fori_loop(
                    0, seq, body,
                    tuple(jnp.zeros((_L,), jnp.float32)
                          for _ in range(dim // _L)),
                )
                for d in range(dim // _L):
                    out_v[c, pl.ds(d * _L, _L)] = accs[d] * inv_seq
            pltpu.sync_copy(out_v, out_hbm.at[pl.ds(row0, cb)])

    return pool_kernel(x_flat, table)


def _tc_head(pooled, w, b2d):
    batch, dim = pooled.shape
    seq = w.shape[0]
    bb = 2048

    def head_kernel(p_ref, w_ref, b_ref, o_ref):
        logits = lax.dot_general(
            p_ref[...], w_ref[...],
            (((1,), (1,)), ((), ())),
            preferred_element_type=jnp.float32,
        ) + b_ref[...]
        o_ref[...] = 1.0 / (1.0 + jnp.exp(-logits))

    return pl.pallas_call(
        head_kernel,
        grid=(batch // bb,),
        in_specs=[
            pl.BlockSpec((bb, dim), lambda i: (i, 0)),
            pl.BlockSpec((seq, dim), lambda i: (0, 0)),
            pl.BlockSpec((1, seq), lambda i: (0, 0)),
        ],
        out_specs=pl.BlockSpec((bb, seq), lambda i: (i, 0)),
        out_shape=jax.ShapeDtypeStruct((batch, seq), jnp.float32),
    )(pooled, w, b2d)


@jax.jit
def kernel(x, table, W, b):
    batch, seq = x.shape
    dim = table.shape[1]
    pooled = _sc_pool(x.reshape(batch * seq), table, batch, seq, dim)
    return _tc_head(pooled, W, b.reshape(1, seq))


# speedup vs baseline: 2.7166x; 1.1733x over previous
import functools

import jax
import jax.numpy as jnp
from jax import lax
from jax.experimental import pallas as pl
from jax.experimental.pallas import tpu as pltpu
from jax.experimental.pallas import tpu_sc as plsc

_NC = 2
_NS = 16
_L = 16


def _sc_pool(x_flat, table, batch, seq, dim):
    nw = _NC * _NS
    rows_per_w = batch // nw
    cb = 8
    steps = rows_per_w // cb
    inv_seq = 1.0 / seq
    nd = dim // _L
    unroll = 10
    assert seq % unroll == 0 and steps % 2 == 0
    mesh = plsc.VectorSubcoreMesh(core_axis_name="c", subcore_axis_name="s")

    @functools.partial(
        pl.kernel,
        mesh=mesh,
        compiler_params=pltpu.CompilerParams(use_tc_tiling_on_sc=False),
        out_type=jax.ShapeDtypeStruct((batch, dim), jnp.float32),
        scratch_types=[
            pltpu.VMEM((rows_per_w * seq,), jnp.int32),
            pltpu.VMEM((cb * seq, dim), jnp.float32),
            pltpu.VMEM((cb * seq, dim), jnp.float32),
            pltpu.VMEM((cb, dim), jnp.float32),
            pltpu.VMEM((cb, dim), jnp.float32),
            pltpu.SemaphoreType.DMA,
            pltpu.SemaphoreType.DMA,
            pltpu.SemaphoreType.DMA,
            pltpu.SemaphoreType.DMA,
        ],
    )
    def pool_kernel(x_hbm, table_hbm, out_hbm, idx_all, rows0, rows1,
                    oacc0, oacc1, sg0, sg1, so0, so1):
        wid = lax.axis_index("s") * _NC + lax.axis_index("c")
        base = wid * rows_per_w

        pltpu.sync_copy(x_hbm.at[pl.ds(base * seq, rows_per_w * seq)],
                        idx_all)
        pltpu.async_copy(
            table_hbm.at[idx_all.at[pl.ds(0, cb * seq)]], rows0, sg0)
        pltpu.async_copy(
            table_hbm.at[idx_all.at[pl.ds(cb * seq, cb * seq)]], rows1, sg1)

        @pl.loop(0, steps, step=2)
        def _(g):
            for slot, rows_v, oacc, sg, so in (
                    (0, rows0, oacc0, sg0, so0),
                    (1, rows1, oacc1, sg1, so1)):
                gg = g + slot
                pltpu.make_async_copy(
                    table_hbm.at[pl.ds(0, cb * seq)], rows_v, sg).wait()
                @pl.when(gg >= 2)
                def _():
                    pltpu.make_async_copy(
                        oacc, out_hbm.at[pl.ds(0, cb)], so).wait()
                for c in range(cb):
                    def body(s, accs, c=c):
                        for j in range(unroll):
                            r = c * seq + s * unroll + j
                            accs = tuple(
                                accs[d] + rows_v[r, pl.ds(d * _L, _L)]
                                for d in range(nd))
                        return accs
                    accs = lax.fori_loop(
                        0, seq // unroll, body,
                        tuple(jnp.zeros((_L,), jnp.float32)
                              for _ in range(nd)))
                    for d in range(nd):
                        oacc[c, pl.ds(d * _L, _L)] = accs[d] * inv_seq
                pltpu.async_copy(
                    oacc, out_hbm.at[pl.ds(base + gg * cb, cb)], so)

                @pl.when(gg + 2 < steps)
                def _():
                    off = (gg + 2) * cb * seq
                    pltpu.async_copy(
                        table_hbm.at[idx_all.at[pl.ds(off, cb * seq)]],
                        rows_v, sg)

        pltpu.make_async_copy(oacc0, out_hbm.at[pl.ds(0, cb)], so0).wait()
        pltpu.make_async_copy(oacc1, out_hbm.at[pl.ds(0, cb)], so1).wait()

    return pool_kernel(x_flat, table)


def _tc_head(pooled, w, b2d):
    batch, dim = pooled.shape
    seq = w.shape[0]
    bb = 2048

    def head_kernel(p_ref, w_ref, b_ref, o_ref):
        logits = lax.dot_general(
            p_ref[...], w_ref[...],
            (((1,), (1,)), ((), ())),
            preferred_element_type=jnp.float32,
        ) + b_ref[...]
        o_ref[...] = 1.0 / (1.0 + jnp.exp(-logits))

    return pl.pallas_call(
        head_kernel,
        grid=(batch // bb,),
        in_specs=[
            pl.BlockSpec((bb, dim), lambda i: (i, 0)),
            pl.BlockSpec((seq, dim), lambda i: (0, 0)),
            pl.BlockSpec((1, seq), lambda i: (0, 0)),
        ],
        out_specs=pl.BlockSpec((bb, seq), lambda i: (i, 0)),
        out_shape=jax.ShapeDtypeStruct((batch, seq), jnp.float32),
    )(pooled, w, b2d)


@jax.jit
def kernel(x, table, W, b):
    batch, seq = x.shape
    dim = table.shape[1]
    pooled = _sc_pool(x.reshape(batch * seq), table, batch, seq, dim)
    return _tc_head(pooled, W, b.reshape(1, seq))
